# baseline (speedup 1.0000x reference)
import functools

import jax
import jax.numpy as jnp
from jax import lax
from jax.experimental import pallas as pl
from jax.experimental.pallas import tpu as pltpu
from jax.experimental.pallas import tpu_sc as plsc

EPS = 0.01
ITP = 10
PT = 512
NBLK = 2048

SC_NC = 2
SC_NS = 16
SC_LANES = 16
NW = SC_NC * SC_NS


def _knn_body(qn2_ref, q2_ref, sT_ref, s2_ref, idx_ref, d2_ref):
    b = pl.program_id(0)
    n = sT_ref.shape[-1]
    pt = q2_ref.shape[-1]

    qm = qn2_ref[0].astype(jnp.bfloat16)
    q2c = q2_ref[0, 0, :].reshape(pt, 1)

    for nb in range(n // NBLK):
        sl = pl.ds(nb * NBLK, NBLK)
        st = sT_ref[0, :, sl].astype(jnp.bfloat16)
        s2b = s2_ref[0, 0, sl].reshape(1, NBLK)
        qs2 = jax.lax.dot_general(qm, st, (((1,), (0,)), ((), ())),
                                  preferred_element_type=jnp.float32)
        d2_ref[:, sl] = (q2c + s2b) + qs2
    bidx = jnp.argmin(d2_ref[...], axis=1).astype(jnp.int32)
    idx_ref[0, 0, :] = bidx + b * n


def _sc_loss_body(q_h, s_h, n_h, idx_h, out_h, idx_v, q_v, s_v, n_v, acc_v):
    qpw = idx_v.shape[0]
    n3 = s_v.shape[0]
    wid = lax.axis_index("s") * SC_NC + lax.axis_index("c")
    base = wid * qpw
    boff3 = (wid // SC_NS) * n3
    pltpu.sync_copy(idx_h.at[pl.ds(base, qpw)], idx_v)
    pltpu.sync_copy(q_h.at[pl.ds(base * 3, qpw * 3)], q_v)
    pltpu.sync_copy(s_h.at[pl.ds(boff3, n3)], s_v)
    pltpu.sync_copy(n_h.at[pl.ds(boff3, n3)], n_v)

    lane3 = jnp.arange(SC_LANES, dtype=jnp.int32) * 3
    acc = jnp.zeros((SC_LANES,), dtype=jnp.float32)
    for g in range(qpw // SC_LANES):
        ii = idx_v[pl.ds(g * SC_LANES, SC_LANES)]
        i3 = ii * 3 - boff3
        gnx = plsc.load_gather(n_v, [i3])
        gny = plsc.load_gather(n_v, [i3 + 1])
        gnz = plsc.load_gather(n_v, [i3 + 2])
        gsx = plsc.load_gather(s_v, [i3])
        gsy = plsc.load_gather(s_v, [i3 + 1])
        gsz = plsc.load_gather(s_v, [i3 + 2])
        q3 = lane3 + (g * SC_LANES * 3)
        qx = plsc.load_gather(q_v, [q3])
        qy = plsc.load_gather(q_v, [q3 + 1])
        qz = plsc.load_gather(q_v, [q3 + 2])
        dot = ((qx * gnx + qy * gny + qz * gnz)
               - (gsx * gnx + gsy * gny + gsz * gnz)
               - EPS * (gnx * gnx + gny * gny + gnz * gnz))
        acc = acc + jnp.maximum(-dot, 0.0)
    acc_v[...] = acc
    pltpu.sync_copy(acc_v, out_h.at[pl.ds(wid * SC_LANES, SC_LANES)])


def kernel(cage, shape, shape_normals):
    B, M, D = cage.shape
    N = shape.shape[1]
    P = M * ITP

    cage_p = jnp.roll(cage, shift=-1, axis=1)
    t = jnp.linspace(0.0, 1.0, ITP).reshape(1, 1, ITP, 1)
    q = (t * cage_p[:, :, None, :] + (1.0 - t) * cage[:, :, None, :])
    q = q.reshape(B, P, D)

    qn2 = -2.0 * q
    q2 = jnp.sum(q * q, axis=-1).reshape(B, 1, P)
    sT = shape.transpose(0, 2, 1)
    s2 = jnp.sum(shape * shape, axis=-1).reshape(B, 1, N)

    nn_idx = pl.pallas_call(
        _knn_body,
        grid=(B, P // PT),
        in_specs=[pl.BlockSpec((1, PT, 3), lambda b, j: (b, j, 0)),
                  pl.BlockSpec((1, 1, PT), lambda b, j: (b, 0, j)),
                  pl.BlockSpec((1, 3, N), lambda b, j: (b, 0, 0)),
                  pl.BlockSpec((1, 1, N), lambda b, j: (b, 0, 0))],
        out_specs=pl.BlockSpec((1, 1, PT), lambda b, j: (b, 0, j)),
        out_shape=jax.ShapeDtypeStruct((B, 1, P), jnp.int32),
        scratch_shapes=[pltpu.VMEM((PT, N), jnp.float32)],
    )(qn2, q2, sT, s2)

    qpw = (B * P) // NW
    sc_loss = functools.partial(
        pl.kernel,
        out_type=jax.ShapeDtypeStruct((NW * SC_LANES,), jnp.float32),
        mesh=plsc.VectorSubcoreMesh(core_axis_name="c", subcore_axis_name="s"),
        scratch_types=[
            pltpu.VMEM((qpw,), jnp.int32),
            pltpu.VMEM((qpw * 3,), jnp.float32),
            pltpu.VMEM((N * 3,), jnp.float32),
            pltpu.VMEM((N * 3,), jnp.float32),
            pltpu.VMEM((SC_LANES,), jnp.float32),
        ],
        compiler_params=pltpu.CompilerParams(needs_layout_passes=False),
    )(_sc_loss_body)

    partials = sc_loss(
        q.reshape(B * P * D), shape.reshape(B * N * D),
        shape_normals.reshape(B * N * D), nn_idx.reshape(B * P))
    return jnp.sum(partials) / (B * P)


# speedup vs baseline: 1.0139x; 1.0139x over previous
import functools

import jax
import jax.numpy as jnp
from jax import lax
from jax.experimental import pallas as pl
from jax.experimental.pallas import tpu as pltpu
from jax.experimental.pallas import tpu_sc as plsc

EPS = 0.01
ITP = 10
PT = 512
NBLK = 2048

SC_NC = 2
SC_NS = 16
SC_LANES = 16
NW = SC_NC * SC_NS


def _knn_body(qn2_ref, q2_ref, sT_ref, s2_ref, idx_ref, d2_ref):
    b = pl.program_id(0)
    n = sT_ref.shape[-1]
    pt = q2_ref.shape[-1]

    qm = qn2_ref[0].astype(jnp.bfloat16)
    q2c = q2_ref[0, 0, :].reshape(pt, 1)

    for nb in range(n // NBLK):
        sl = pl.ds(nb * NBLK, NBLK)
        st = sT_ref[0, :, sl].astype(jnp.bfloat16)
        s2b = s2_ref[0, 0, sl].reshape(1, NBLK)
        qs2 = jax.lax.dot_general(qm, st, (((1,), (0,)), ((), ())),
                                  preferred_element_type=jnp.float32)
        d2_ref[:, sl] = (q2c + s2b) + qs2
    bidx = jnp.argmin(d2_ref[...], axis=1).astype(jnp.int32)
    idx_ref[0, 0, :] = bidx + b * n


def _sc_loss_body(q_h, s_h, n_h, idx_h, out_h, idx_v, q_v, s_v, n_v, acc_v,
                  sem):
    qpw = idx_v.shape[0]
    n3 = s_v.shape[0]
    wid = lax.axis_index("s") * SC_NC + lax.axis_index("c")
    base = wid * qpw
    boff3 = (wid // SC_NS) * n3
    cps = [pltpu.async_copy(idx_h.at[pl.ds(base, qpw)], idx_v, sem),
           pltpu.async_copy(q_h.at[pl.ds(base * 3, qpw * 3)], q_v, sem),
           pltpu.async_copy(s_h.at[pl.ds(boff3, n3)], s_v, sem),
           pltpu.async_copy(n_h.at[pl.ds(boff3, n3)], n_v, sem)]
    for cp in cps:
        cp.wait()

    lane3 = jnp.arange(SC_LANES, dtype=jnp.int32) * 3
    acc = jnp.zeros((SC_LANES,), dtype=jnp.float32)
    for g in range(qpw // SC_LANES):
        ii = idx_v[pl.ds(g * SC_LANES, SC_LANES)]
        i3 = ii * 3 - boff3
        gnx = plsc.load_gather(n_v, [i3])
        gny = plsc.load_gather(n_v, [i3 + 1])
        gnz = plsc.load_gather(n_v, [i3 + 2])
        gsx = plsc.load_gather(s_v, [i3])
        gsy = plsc.load_gather(s_v, [i3 + 1])
        gsz = plsc.load_gather(s_v, [i3 + 2])
        q3 = lane3 + (g * SC_LANES * 3)
        qx = plsc.load_gather(q_v, [q3])
        qy = plsc.load_gather(q_v, [q3 + 1])
        qz = plsc.load_gather(q_v, [q3 + 2])
        dot = ((qx * gnx + qy * gny + qz * gnz)
               - (gsx * gnx + gsy * gny + gsz * gnz)
               - EPS * (gnx * gnx + gny * gny + gnz * gnz))
        acc = acc + jnp.maximum(-dot, 0.0)
    acc_v[...] = acc
    pltpu.sync_copy(acc_v, out_h.at[pl.ds(wid * SC_LANES, SC_LANES)])


def kernel(cage, shape, shape_normals):
    B, M, D = cage.shape
    N = shape.shape[1]
    P = M * ITP

    cage_p = jnp.roll(cage, shift=-1, axis=1)
    t = jnp.linspace(0.0, 1.0, ITP).reshape(1, 1, ITP, 1)
    q = (t * cage_p[:, :, None, :] + (1.0 - t) * cage[:, :, None, :])
    q = q.reshape(B, P, D)

    qn2 = -2.0 * q
    q2 = jnp.sum(q * q, axis=-1).reshape(B, 1, P)
    sT = shape.transpose(0, 2, 1)
    s2 = jnp.sum(shape * shape, axis=-1).reshape(B, 1, N)

    nn_idx = pl.pallas_call(
        _knn_body,
        grid=(B, P // PT),
        in_specs=[pl.BlockSpec((1, PT, 3), lambda b, j: (b, j, 0)),
                  pl.BlockSpec((1, 1, PT), lambda b, j: (b, 0, j)),
                  pl.BlockSpec((1, 3, N), lambda b, j: (b, 0, 0)),
                  pl.BlockSpec((1, 1, N), lambda b, j: (b, 0, 0))],
        out_specs=pl.BlockSpec((1, 1, PT), lambda b, j: (b, 0, j)),
        out_shape=jax.ShapeDtypeStruct((B, 1, P), jnp.int32),
        scratch_shapes=[pltpu.VMEM((PT, N), jnp.float32)],
    )(qn2, q2, sT, s2)

    qpw = (B * P) // NW
    sc_loss = functools.partial(
        pl.kernel,
        out_type=jax.ShapeDtypeStruct((NW * SC_LANES,), jnp.float32),
        mesh=plsc.VectorSubcoreMesh(core_axis_name="c", subcore_axis_name="s"),
        scratch_types=[
            pltpu.VMEM((qpw,), jnp.int32),
            pltpu.VMEM((qpw * 3,), jnp.float32),
            pltpu.VMEM((N * 3,), jnp.float32),
            pltpu.VMEM((N * 3,), jnp.float32),
            pltpu.VMEM((SC_LANES,), jnp.float32),
            pltpu.SemaphoreType.DMA,
        ],
        compiler_params=pltpu.CompilerParams(needs_layout_passes=False),
    )(_sc_loss_body)

    partials = sc_loss(
        q.reshape(B * P * D), shape.reshape(B * N * D),
        shape_normals.reshape(B * N * D), nn_idx.reshape(B * P))
    return jnp.sum(partials) / (B * P)


# speedup vs baseline: 1.4040x; 1.3848x over previous
import functools

import jax
import jax.numpy as jnp
from jax import lax
from jax.experimental import pallas as pl
from jax.experimental.pallas import tpu as pltpu
from jax.experimental.pallas import tpu_sc as plsc

EPS = 0.01
ITP = 10
PT = 512
NBLK = 2048

SC_NC = 2
SC_NS = 16
SC_LANES = 16
NW = SC_NC * SC_NS


def _knn_body(qn2_ref, q2_ref, sT_ref, s2_ref, nT_ref, idx_ref, tbl_ref,
              d2_ref):
    b = pl.program_id(0)
    n = sT_ref.shape[-1]
    pt = q2_ref.shape[-1]

    sxr = sT_ref[0, 0, :]
    syr = sT_ref[0, 1, :]
    szr = sT_ref[0, 2, :]
    nxr = nT_ref[0, 0, :]
    nyr = nT_ref[0, 1, :]
    nzr = nT_ref[0, 2, :]
    tbl_ref[0, 0, :] = nxr
    tbl_ref[0, 1, :] = nyr
    tbl_ref[0, 2, :] = nzr
    tbl_ref[0, 3, :] = (sxr * nxr + syr * nyr + szr * nzr
                        + EPS * (nxr * nxr + nyr * nyr + nzr * nzr))

    qm = qn2_ref[0].astype(jnp.bfloat16)
    q2c = q2_ref[0, 0, :].reshape(pt, 1)

    for nb in range(n // NBLK):
        sl = pl.ds(nb * NBLK, NBLK)
        st = sT_ref[0, :, sl].astype(jnp.bfloat16)
        s2b = s2_ref[0, 0, sl].reshape(1, NBLK)
        qs2 = jax.lax.dot_general(qm, st, (((1,), (0,)), ((), ())),
                                  preferred_element_type=jnp.float32)
        d2_ref[:, sl] = (q2c + s2b) + qs2
    bidx = jnp.argmin(d2_ref[...], axis=1).astype(jnp.int32)
    idx_ref[0, 0, :] = bidx + b * n


def _sc_loss_body(q_h, tbl_h, idx_h, out_h, idx_v, q_v, t_v, acc_v, sem):
    qpw = q_v.shape[0]
    n1 = t_v.shape[-1]
    wid = lax.axis_index("s") * SC_NC + lax.axis_index("c")
    batch = wid // SC_NS
    qb = (wid % SC_NS) * qpw
    boff = batch * n1
    cps = [pltpu.async_copy(idx_h.at[batch, 0], idx_v, sem),
           pltpu.async_copy(q_h.at[batch, pl.ds(qb, qpw)], q_v, sem),
           pltpu.async_copy(tbl_h.at[batch], t_v, sem)]
    for cp in cps:
        cp.wait()

    lane = jnp.arange(SC_LANES, dtype=jnp.int32)
    c0 = jnp.full((SC_LANES,), 0, jnp.int32)
    c1 = jnp.full((SC_LANES,), 1, jnp.int32)
    c2 = jnp.full((SC_LANES,), 2, jnp.int32)
    c3 = jnp.full((SC_LANES,), 3, jnp.int32)
    acc = jnp.zeros((SC_LANES,), dtype=jnp.float32)
    for g in range(qpw // SC_LANES):
        ii = plsc.load_gather(idx_v, [lane + (qb + g * SC_LANES)])
        il = ii - boff
        gnx = plsc.load_gather(t_v, [c0, il])
        gny = plsc.load_gather(t_v, [c1, il])
        gnz = plsc.load_gather(t_v, [c2, il])
        gc = plsc.load_gather(t_v, [c3, il])
        ql = lane + g * SC_LANES
        qx = plsc.load_gather(q_v, [ql, c0])
        qy = plsc.load_gather(q_v, [ql, c1])
        qz = plsc.load_gather(q_v, [ql, c2])
        dot = (qx * gnx + qy * gny + qz * gnz) - gc
        acc = acc + jnp.maximum(-dot, 0.0)
    acc_v[...] = acc
    pltpu.sync_copy(acc_v, out_h.at[pl.ds(wid * SC_LANES, SC_LANES)])


def kernel(cage, shape, shape_normals):
    B, M, D = cage.shape
    N = shape.shape[1]
    P = M * ITP

    cage_p = jnp.roll(cage, shift=-1, axis=1)
    t = jnp.linspace(0.0, 1.0, ITP).reshape(1, 1, ITP, 1)
    q = (t * cage_p[:, :, None, :] + (1.0 - t) * cage[:, :, None, :])
    q = q.reshape(B, P, D)

    qn2 = -2.0 * q
    q2 = jnp.sum(q * q, axis=-1).reshape(B, 1, P)
    sT = shape.transpose(0, 2, 1)
    s2 = jnp.sum(shape * shape, axis=-1).reshape(B, 1, N)

    nT = shape_normals.transpose(0, 2, 1)

    nn_idx, tbl = pl.pallas_call(
        _knn_body,
        grid=(B, P // PT),
        in_specs=[pl.BlockSpec((1, PT, 3), lambda b, j: (b, j, 0)),
                  pl.BlockSpec((1, 1, PT), lambda b, j: (b, 0, j)),
                  pl.BlockSpec((1, 3, N), lambda b, j: (b, 0, 0)),
                  pl.BlockSpec((1, 1, N), lambda b, j: (b, 0, 0)),
                  pl.BlockSpec((1, 3, N), lambda b, j: (b, 0, 0))],
        out_specs=[pl.BlockSpec((1, 1, PT), lambda b, j: (b, 0, j)),
                   pl.BlockSpec((1, 4, N), lambda b, j: (b, 0, 0))],
        out_shape=[jax.ShapeDtypeStruct((B, 1, P), jnp.int32),
                   jax.ShapeDtypeStruct((B, 4, N), jnp.float32)],
        scratch_shapes=[pltpu.VMEM((PT, N), jnp.float32)],
    )(qn2, q2, sT, s2, nT)

    qpw = (B * P) // NW
    sc_loss = functools.partial(
        pl.kernel,
        out_type=jax.ShapeDtypeStruct((NW * SC_LANES,), jnp.float32),
        mesh=plsc.VectorSubcoreMesh(core_axis_name="c", subcore_axis_name="s"),
        scratch_types=[
            pltpu.VMEM((P,), jnp.int32),
            pltpu.VMEM((qpw, 3), jnp.float32),
            pltpu.VMEM((4, N), jnp.float32),
            pltpu.VMEM((SC_LANES,), jnp.float32),
            pltpu.SemaphoreType.DMA,
        ],
        compiler_params=pltpu.CompilerParams(needs_layout_passes=False),
    )(_sc_loss_body)

    partials = sc_loss(q, tbl, nn_idx)
    return jnp.sum(partials) / (B * P)


# speedup vs baseline: 1.4280x; 1.0171x over previous
import functools

import jax
import jax.numpy as jnp
from jax import lax
from jax.experimental import pallas as pl
from jax.experimental.pallas import tpu as pltpu
from jax.experimental.pallas import tpu_sc as plsc

EPS = 0.01
ITP = 10
PT = 640
NBLK = 2048

SC_NC = 2
SC_NS = 16
SC_LANES = 16
NW = SC_NC * SC_NS


def _knn_body(qn2_ref, q2_ref, sT_ref, s2_ref, nT_ref, idx_ref, tbl_ref,
              d2_ref):
    b = pl.program_id(0)
    n = sT_ref.shape[-1]
    pt = q2_ref.shape[-1]

    sxr = sT_ref[0, 0, :]
    syr = sT_ref[0, 1, :]
    szr = sT_ref[0, 2, :]
    nxr = nT_ref[0, 0, :]
    nyr = nT_ref[0, 1, :]
    nzr = nT_ref[0, 2, :]
    tbl_ref[0, 0, :] = nxr
    tbl_ref[0, 1, :] = nyr
    tbl_ref[0, 2, :] = nzr
    tbl_ref[0, 3, :] = (sxr * nxr + syr * nyr + szr * nzr
                        + EPS * (nxr * nxr + nyr * nyr + nzr * nzr))

    qm = qn2_ref[0].astype(jnp.bfloat16)
    q2c = q2_ref[0, 0, :].reshape(pt, 1)

    for nb in range(n // NBLK):
        sl = pl.ds(nb * NBLK, NBLK)
        st = sT_ref[0, :, sl].astype(jnp.bfloat16)
        s2b = s2_ref[0, 0, sl].reshape(1, NBLK)
        qs2 = jax.lax.dot_general(qm, st, (((1,), (0,)), ((), ())),
                                  preferred_element_type=jnp.float32)
        d2_ref[:, sl] = (q2c + s2b) + qs2
    bidx = jnp.argmin(d2_ref[...], axis=1).astype(jnp.int32)
    idx_ref[0, 0, :] = bidx + b * n


def _sc_loss_body(q_h, tbl_h, idx_h, out_h, idx_v, q_v, t_v, acc_v, sem):
    qpw = q_v.shape[0]
    n1 = t_v.shape[-1]
    wid = lax.axis_index("s") * SC_NC + lax.axis_index("c")
    batch = wid // SC_NS
    qb = (wid % SC_NS) * qpw
    boff = batch * n1
    cps = [pltpu.async_copy(idx_h.at[batch, 0], idx_v, sem),
           pltpu.async_copy(q_h.at[batch, pl.ds(qb, qpw)], q_v, sem),
           pltpu.async_copy(tbl_h.at[batch], t_v, sem)]
    for cp in cps:
        cp.wait()

    lane = jnp.arange(SC_LANES, dtype=jnp.int32)
    c0 = jnp.full((SC_LANES,), 0, jnp.int32)
    c1 = jnp.full((SC_LANES,), 1, jnp.int32)
    c2 = jnp.full((SC_LANES,), 2, jnp.int32)
    c3 = jnp.full((SC_LANES,), 3, jnp.int32)
    acc = jnp.zeros((SC_LANES,), dtype=jnp.float32)
    for g in range(qpw // SC_LANES):
        ii = plsc.load_gather(idx_v, [lane + (qb + g * SC_LANES)])
        il = ii - boff
        gnx = plsc.load_gather(t_v, [c0, il])
        gny = plsc.load_gather(t_v, [c1, il])
        gnz = plsc.load_gather(t_v, [c2, il])
        gc = plsc.load_gather(t_v, [c3, il])
        ql = lane + g * SC_LANES
        qx = plsc.load_gather(q_v, [ql, c0])
        qy = plsc.load_gather(q_v, [ql, c1])
        qz = plsc.load_gather(q_v, [ql, c2])
        dot = (qx * gnx + qy * gny + qz * gnz) - gc
        acc = acc + jnp.maximum(-dot, 0.0)
    acc_v[...] = acc
    pltpu.sync_copy(acc_v, out_h.at[pl.ds(wid * SC_LANES, SC_LANES)])


def kernel(cage, shape, shape_normals):
    B, M, D = cage.shape
    N = shape.shape[1]
    P = M * ITP

    cage_p = jnp.roll(cage, shift=-1, axis=1)
    t = jnp.linspace(0.0, 1.0, ITP).reshape(1, 1, ITP, 1)
    q = (t * cage_p[:, :, None, :] + (1.0 - t) * cage[:, :, None, :])
    q = q.reshape(B, P, D)

    qn2 = -2.0 * q
    q2 = jnp.sum(q * q, axis=-1).reshape(B, 1, P)
    sT = shape.transpose(0, 2, 1)
    s2 = jnp.sum(shape * shape, axis=-1).reshape(B, 1, N)

    nT = shape_normals.transpose(0, 2, 1)

    nn_idx, tbl = pl.pallas_call(
        _knn_body,
        grid=(B, P // PT),
        in_specs=[pl.BlockSpec((1, PT, 3), lambda b, j: (b, j, 0)),
                  pl.BlockSpec((1, 1, PT), lambda b, j: (b, 0, j)),
                  pl.BlockSpec((1, 3, N), lambda b, j: (b, 0, 0)),
                  pl.BlockSpec((1, 1, N), lambda b, j: (b, 0, 0)),
                  pl.BlockSpec((1, 3, N), lambda b, j: (b, 0, 0))],
        out_specs=[pl.BlockSpec((1, 1, PT), lambda b, j: (b, 0, j)),
                   pl.BlockSpec((1, 4, N), lambda b, j: (b, 0, 0))],
        out_shape=[jax.ShapeDtypeStruct((B, 1, P), jnp.int32),
                   jax.ShapeDtypeStruct((B, 4, N), jnp.float32)],
        scratch_shapes=[pltpu.VMEM((PT, N), jnp.float32)],
    )(qn2, q2, sT, s2, nT)

    qpw = (B * P) // NW
    sc_loss = functools.partial(
        pl.kernel,
        out_type=jax.ShapeDtypeStruct((NW * SC_LANES,), jnp.float32),
        mesh=plsc.VectorSubcoreMesh(core_axis_name="c", subcore_axis_name="s"),
        scratch_types=[
            pltpu.VMEM((P,), jnp.int32),
            pltpu.VMEM((qpw, 3), jnp.float32),
            pltpu.VMEM((4, N), jnp.float32),
            pltpu.VMEM((SC_LANES,), jnp.float32),
            pltpu.SemaphoreType.DMA,
        ],
        compiler_params=pltpu.CompilerParams(needs_layout_passes=False),
    )(_sc_loss_body)

    partials = sc_loss(q, tbl, nn_idx)
    return jnp.sum(partials) / (B * P)


# speedup vs baseline: 1.4343x; 1.0045x over previous
import functools

import jax
import jax.numpy as jnp
from jax import lax
from jax.experimental import pallas as pl
from jax.experimental.pallas import tpu as pltpu
from jax.experimental.pallas import tpu_sc as plsc

EPS = 0.01
ITP = 10
PT = 1280
NBLK = 2048

SC_NC = 2
SC_NS = 16
SC_LANES = 16
NW = SC_NC * SC_NS


def _knn_body(qn2_ref, q2_ref, sT_ref, s2_ref, nT_ref, idx_ref, tbl_ref,
              d2_ref):
    b = pl.program_id(0)
    n = sT_ref.shape[-1]
    pt = q2_ref.shape[-1]

    sxr = sT_ref[0, 0, :]
    syr = sT_ref[0, 1, :]
    szr = sT_ref[0, 2, :]
    nxr = nT_ref[0, 0, :]
    nyr = nT_ref[0, 1, :]
    nzr = nT_ref[0, 2, :]
    tbl_ref[0, 0, :] = nxr
    tbl_ref[0, 1, :] = nyr
    tbl_ref[0, 2, :] = nzr
    tbl_ref[0, 3, :] = (sxr * nxr + syr * nyr + szr * nzr
                        + EPS * (nxr * nxr + nyr * nyr + nzr * nzr))

    qm = qn2_ref[0].astype(jnp.bfloat16)
    q2c = q2_ref[0, 0, :].reshape(pt, 1)

    for nb in range(n // NBLK):
        sl = pl.ds(nb * NBLK, NBLK)
        st = sT_ref[0, :, sl].astype(jnp.bfloat16)
        s2b = s2_ref[0, 0, sl].reshape(1, NBLK)
        qs2 = jax.lax.dot_general(qm, st, (((1,), (0,)), ((), ())),
                                  preferred_element_type=jnp.float32)
        d2_ref[:, sl] = (q2c + s2b) + qs2
    bidx = jnp.argmin(d2_ref[...], axis=1).astype(jnp.int32)
    idx_ref[0, 0, :] = bidx + b * n


def _sc_loss_body(q_h, tbl_h, idx_h, out_h, idx_v, q_v, t_v, acc_v, sem):
    qpw = q_v.shape[0]
    n1 = t_v.shape[-1]
    wid = lax.axis_index("s") * SC_NC + lax.axis_index("c")
    batch = wid // SC_NS
    qb = (wid % SC_NS) * qpw
    boff = batch * n1
    cps = [pltpu.async_copy(idx_h.at[batch, 0], idx_v, sem),
           pltpu.async_copy(q_h.at[batch, pl.ds(qb, qpw)], q_v, sem),
           pltpu.async_copy(tbl_h.at[batch], t_v, sem)]
    for cp in cps:
        cp.wait()

    lane = jnp.arange(SC_LANES, dtype=jnp.int32)
    c0 = jnp.full((SC_LANES,), 0, jnp.int32)
    c1 = jnp.full((SC_LANES,), 1, jnp.int32)
    c2 = jnp.full((SC_LANES,), 2, jnp.int32)
    c3 = jnp.full((SC_LANES,), 3, jnp.int32)
    acc = jnp.zeros((SC_LANES,), dtype=jnp.float32)
    for g in range(qpw // SC_LANES):
        ii = plsc.load_gather(idx_v, [lane + (qb + g * SC_LANES)])
        il = ii - boff
        gnx = plsc.load_gather(t_v, [c0, il])
        gny = plsc.load_gather(t_v, [c1, il])
        gnz = plsc.load_gather(t_v, [c2, il])
        gc = plsc.load_gather(t_v, [c3, il])
        ql = lane + g * SC_LANES
        qx = plsc.load_gather(q_v, [ql, c0])
        qy = plsc.load_gather(q_v, [ql, c1])
        qz = plsc.load_gather(q_v, [ql, c2])
        dot = (qx * gnx + qy * gny + qz * gnz) - gc
        acc = acc + jnp.maximum(-dot, 0.0)
    acc_v[...] = acc
    pltpu.sync_copy(acc_v, out_h.at[pl.ds(wid * SC_LANES, SC_LANES)])


def kernel(cage, shape, shape_normals):
    B, M, D = cage.shape
    N = shape.shape[1]
    P = M * ITP

    cage_p = jnp.roll(cage, shift=-1, axis=1)
    t = jnp.linspace(0.0, 1.0, ITP).reshape(1, 1, ITP, 1)
    q = (t * cage_p[:, :, None, :] + (1.0 - t) * cage[:, :, None, :])
    q = q.reshape(B, P, D)

    qn2 = -2.0 * q
    q2 = jnp.sum(q * q, axis=-1).reshape(B, 1, P)
    sT = shape.transpose(0, 2, 1)
    s2 = jnp.sum(shape * shape, axis=-1).reshape(B, 1, N)

    nT = shape_normals.transpose(0, 2, 1)

    nn_idx, tbl = pl.pallas_call(
        _knn_body,
        grid=(B, P // PT),
        in_specs=[pl.BlockSpec((1, PT, 3), lambda b, j: (b, j, 0)),
                  pl.BlockSpec((1, 1, PT), lambda b, j: (b, 0, j)),
                  pl.BlockSpec((1, 3, N), lambda b, j: (b, 0, 0)),
                  pl.BlockSpec((1, 1, N), lambda b, j: (b, 0, 0)),
                  pl.BlockSpec((1, 3, N), lambda b, j: (b, 0, 0))],
        out_specs=[pl.BlockSpec((1, 1, PT), lambda b, j: (b, 0, j)),
                   pl.BlockSpec((1, 4, N), lambda b, j: (b, 0, 0))],
        out_shape=[jax.ShapeDtypeStruct((B, 1, P), jnp.int32),
                   jax.ShapeDtypeStruct((B, 4, N), jnp.float32)],
        scratch_shapes=[pltpu.VMEM((PT, N), jnp.float32)],
    )(qn2, q2, sT, s2, nT)

    qpw = (B * P) // NW
    sc_loss = functools.partial(
        pl.kernel,
        out_type=jax.ShapeDtypeStruct((NW * SC_LANES,), jnp.float32),
        mesh=plsc.VectorSubcoreMesh(core_axis_name="c", subcore_axis_name="s"),
        scratch_types=[
            pltpu.VMEM((P,), jnp.int32),
            pltpu.VMEM((qpw, 3), jnp.float32),
            pltpu.VMEM((4, N), jnp.float32),
            pltpu.VMEM((SC_LANES,), jnp.float32),
            pltpu.SemaphoreType.DMA,
        ],
        compiler_params=pltpu.CompilerParams(needs_layout_passes=False),
    )(_sc_loss_body)

    partials = sc_loss(q, tbl, nn_idx)
    return jnp.sum(partials) / (B * P)


# speedup vs baseline: 1.4349x; 1.0004x over previous
import functools

import jax
import jax.numpy as jnp
from jax import lax
from jax.experimental import pallas as pl
from jax.experimental.pallas import tpu as pltpu
from jax.experimental.pallas import tpu_sc as plsc

EPS = 0.01
ITP = 10
PT = 1280
NBLK = 4096

SC_NC = 2
SC_NS = 16
SC_LANES = 16
NW = SC_NC * SC_NS


def _knn_body(qn2_ref, q2_ref, sT_ref, s2_ref, nT_ref, idx_ref, tbl_ref,
              d2_ref):
    b = pl.program_id(0)
    n = sT_ref.shape[-1]
    pt = q2_ref.shape[-1]

    sxr = sT_ref[0, 0, :]
    syr = sT_ref[0, 1, :]
    szr = sT_ref[0, 2, :]
    nxr = nT_ref[0, 0, :]
    nyr = nT_ref[0, 1, :]
    nzr = nT_ref[0, 2, :]
    tbl_ref[0, 0, :] = nxr
    tbl_ref[0, 1, :] = nyr
    tbl_ref[0, 2, :] = nzr
    tbl_ref[0, 3, :] = (sxr * nxr + syr * nyr + szr * nzr
                        + EPS * (nxr * nxr + nyr * nyr + nzr * nzr))

    qm = qn2_ref[0].astype(jnp.bfloat16)
    q2c = q2_ref[0, 0, :].reshape(pt, 1)

    for nb in range(n // NBLK):
        sl = pl.ds(nb * NBLK, NBLK)
        st = sT_ref[0, :, sl].astype(jnp.bfloat16)
        s2b = s2_ref[0, 0, sl].reshape(1, NBLK)
        qs2 = jax.lax.dot_general(qm, st, (((1,), (0,)), ((), ())),
                                  preferred_element_type=jnp.float32)
        d2_ref[:, sl] = (q2c + s2b) + qs2
    bidx = jnp.argmin(d2_ref[...], axis=1).astype(jnp.int32)
    idx_ref[0, 0, :] = bidx + b * n


def _sc_loss_body(q_h, tbl_h, idx_h, out_h, idx_v, q_v, t_v, acc_v, sem):
    qpw = q_v.shape[0]
    n1 = t_v.shape[-1]
    wid = lax.axis_index("s") * SC_NC + lax.axis_index("c")
    batch = wid // SC_NS
    qb = (wid % SC_NS) * qpw
    boff = batch * n1
    cps = [pltpu.async_copy(idx_h.at[batch, 0], idx_v, sem),
           pltpu.async_copy(q_h.at[batch, pl.ds(qb, qpw)], q_v, sem),
           pltpu.async_copy(tbl_h.at[batch], t_v, sem)]
    for cp in cps:
        cp.wait()

    lane = jnp.arange(SC_LANES, dtype=jnp.int32)
    c0 = jnp.full((SC_LANES,), 0, jnp.int32)
    c1 = jnp.full((SC_LANES,), 1, jnp.int32)
    c2 = jnp.full((SC_LANES,), 2, jnp.int32)
    c3 = jnp.full((SC_LANES,), 3, jnp.int32)
    acc = jnp.zeros((SC_LANES,), dtype=jnp.float32)
    for g in range(qpw // SC_LANES):
        ii = plsc.load_gather(idx_v, [lane + (qb + g * SC_LANES)])
        il = ii - boff
        gnx = plsc.load_gather(t_v, [c0, il])
        gny = plsc.load_gather(t_v, [c1, il])
        gnz = plsc.load_gather(t_v, [c2, il])
        gc = plsc.load_gather(t_v, [c3, il])
        ql = lane + g * SC_LANES
        qx = plsc.load_gather(q_v, [ql, c0])
        qy = plsc.load_gather(q_v, [ql, c1])
        qz = plsc.load_gather(q_v, [ql, c2])
        dot = (qx * gnx + qy * gny + qz * gnz) - gc
        acc = acc + jnp.maximum(-dot, 0.0)
    acc_v[...] = acc
    pltpu.sync_copy(acc_v, out_h.at[pl.ds(wid * SC_LANES, SC_LANES)])


def kernel(cage, shape, shape_normals):
    B, M, D = cage.shape
    N = shape.shape[1]
    P = M * ITP

    cage_p = jnp.roll(cage, shift=-1, axis=1)
    t = jnp.linspace(0.0, 1.0, ITP).reshape(1, 1, ITP, 1)
    q = (t * cage_p[:, :, None, :] + (1.0 - t) * cage[:, :, None, :])
    q = q.reshape(B, P, D)

    qn2 = -2.0 * q
    q2 = jnp.sum(q * q, axis=-1).reshape(B, 1, P)
    sT = shape.transpose(0, 2, 1)
    s2 = jnp.sum(shape * shape, axis=-1).reshape(B, 1, N)

    nT = shape_normals.transpose(0, 2, 1)

    nn_idx, tbl = pl.pallas_call(
        _knn_body,
        grid=(B, P // PT),
        in_specs=[pl.BlockSpec((1, PT, 3), lambda b, j: (b, j, 0)),
                  pl.BlockSpec((1, 1, PT), lambda b, j: (b, 0, j)),
                  pl.BlockSpec((1, 3, N), lambda b, j: (b, 0, 0)),
                  pl.BlockSpec((1, 1, N), lambda b, j: (b, 0, 0)),
                  pl.BlockSpec((1, 3, N), lambda b, j: (b, 0, 0))],
        out_specs=[pl.BlockSpec((1, 1, PT), lambda b, j: (b, 0, j)),
                   pl.BlockSpec((1, 4, N), lambda b, j: (b, 0, 0))],
        out_shape=[jax.ShapeDtypeStruct((B, 1, P), jnp.int32),
                   jax.ShapeDtypeStruct((B, 4, N), jnp.float32)],
        scratch_shapes=[pltpu.VMEM((PT, N), jnp.float32)],
    )(qn2, q2, sT, s2, nT)

    qpw = (B * P) // NW
    sc_loss = functools.partial(
        pl.kernel,
        out_type=jax.ShapeDtypeStruct((NW * SC_LANES,), jnp.float32),
        mesh=plsc.VectorSubcoreMesh(core_axis_name="c", subcore_axis_name="s"),
        scratch_types=[
            pltpu.VMEM((P,), jnp.int32),
            pltpu.VMEM((qpw, 3), jnp.float32),
            pltpu.VMEM((4, N), jnp.float32),
            pltpu.VMEM((SC_LANES,), jnp.float32),
            pltpu.SemaphoreType.DMA,
        ],
        compiler_params=pltpu.CompilerParams(needs_layout_passes=False),
    )(_sc_loss_body)

    partials = sc_loss(q, tbl, nn_idx)
    return jnp.sum(partials) / (B * P)
